# Initial kernel scaffold; baseline (speedup 1.0000x reference)
import jax
import jax.numpy as jnp
from jax.experimental import pallas as pl


def kernel(x, edge_index, edge_weight, W, bias, gamma, beta, running_mean, running_var):
    raise NotImplementedError("write your pallas kernel here")


# speedup vs baseline: 12.0059x; 12.0059x over previous
import functools

import jax
import jax.numpy as jnp
from jax import lax
from jax.experimental import pallas as pl
from jax.experimental.pallas import tpu as pltpu
from jax.experimental.pallas import tpu_sc as plsc

N = 10000
E = 320000
C = 128

NC = 2
NS = 16
NW = NC * NS
KE = 80
EPW = E // NW
CHUNKS_PER_TILE = EPW // KE
NPAD = 10240
DEG_PER_TILE = NPAD // NS
ACC_PER_TILE = NPAD // NS
DUMP = 128

_mesh = plsc.VectorSubcoreMesh(core_axis_name="c", subcore_axis_name="s")
_sc_params = pltpu.CompilerParams(needs_layout_passes=False)


@functools.partial(
    pl.kernel,
    out_type=jax.ShapeDtypeStruct((NC * NPAD,), jnp.float32),
    mesh=_mesh,
    compiler_params=_sc_params,
    scratch_types=[
        pltpu.VMEM_SHARED((NPAD,), jnp.float32),
        pltpu.VMEM((KE,), jnp.int32),
        pltpu.VMEM((KE,), jnp.float32),
        pltpu.VMEM((DEG_PER_TILE,), jnp.float32),
    ],
)
def _sc_deg(col_hbm, w_hbm, degp_hbm, deg_s, colv, wv, zv):
    c = lax.axis_index("c")
    s = lax.axis_index("s")
    wid = s * NC + c
    z16 = jnp.zeros((16,), jnp.float32)
    for i in range(DEG_PER_TILE // 16):
        zv[pl.ds(i * 16, 16)] = z16
    pltpu.sync_copy(zv, deg_s.at[pl.ds(s * DEG_PER_TILE, DEG_PER_TILE)])
    plsc.subcore_barrier()
    base = wid * EPW

    def chunk_body(j, _):
        e0 = base + j * KE
        pltpu.sync_copy(col_hbm.at[pl.ds(e0, KE)], colv)
        pltpu.sync_copy(w_hbm.at[pl.ds(e0, KE)], wv)
        pltpu.sync_copy(wv, deg_s.at[colv], add=True)
        return 0

    lax.fori_loop(0, CHUNKS_PER_TILE, chunk_body, 0)
    plsc.subcore_barrier()
    pltpu.sync_copy(deg_s.at[pl.ds(s * DEG_PER_TILE, DEG_PER_TILE)], zv)
    pltpu.sync_copy(zv, degp_hbm.at[pl.ds(c * NPAD + s * DEG_PER_TILE, DEG_PER_TILE)])


@functools.partial(
    pl.kernel,
    out_type=jax.ShapeDtypeStruct((NC, NPAD, C), jnp.float32),
    mesh=_mesh,
    compiler_params=_sc_params,
    scratch_types=[
        pltpu.VMEM_SHARED((NPAD, C), jnp.float32),
        pltpu.VMEM((NPAD,), jnp.float32),
        pltpu.VMEM((KE,), jnp.int32),
        pltpu.VMEM((KE,), jnp.int32),
        pltpu.VMEM((KE,), jnp.float32),
        pltpu.VMEM((KE, C), jnp.float32),
        pltpu.VMEM((KE,), jnp.float32),
        pltpu.VMEM((DUMP, C), jnp.float32),
        pltpu.SemaphoreType.DMA,
    ],
)
def _sc_edge(hs_hbm, row_hbm, col_hbm, w_hbm, dinv_hbm, part_hbm,
             acc_s, dinv_v, rowv, colv, wv, rows_v, norm_v, stage_v, sem):
    c = lax.axis_index("c")
    s = lax.axis_index("s")
    wid = s * NC + c
    z16 = jnp.zeros((16,), jnp.float32)

    def zrow(i, _):
        for k8 in range(C // 16):
            stage_v[i, pl.ds(k8 * 16, 16)] = z16
        return 0

    lax.fori_loop(0, DUMP, zrow, 0)
    for q in range(ACC_PER_TILE // DUMP):
        pltpu.sync_copy(stage_v, acc_s.at[pl.ds(s * ACC_PER_TILE + q * DUMP, DUMP), :])
    pltpu.sync_copy(dinv_hbm, dinv_v)
    plsc.subcore_barrier()
    base = wid * EPW

    def chunk_body(j, _):
        e0 = base + j * KE
        pltpu.sync_copy(row_hbm.at[pl.ds(e0, KE)], rowv)
        pltpu.sync_copy(col_hbm.at[pl.ds(e0, KE)], colv)
        pltpu.sync_copy(w_hbm.at[pl.ds(e0, KE)], wv)
        cp = pltpu.make_async_copy(hs_hbm.at[rowv], rows_v, sem)
        cp.start()
        for t in range(KE // 16):
            c16 = colv[pl.ds(t * 16, 16)]
            w16 = wv[pl.ds(t * 16, 16)]
            norm_v[pl.ds(t * 16, 16)] = plsc.load_gather(dinv_v, [c16]) * w16
        cp.wait()

        def scale_body(e, _):
            sc = plsc.load_gather(norm_v, [jnp.full((16,), e, jnp.int32)])
            for k8 in range(C // 16):
                v = rows_v[e, pl.ds(k8 * 16, 16)]
                rows_v[e, pl.ds(k8 * 16, 16)] = v * sc
            return 0

        lax.fori_loop(0, KE, scale_body, 0)
        pltpu.sync_copy(rows_v, acc_s.at[colv], add=True)
        return 0

    lax.fori_loop(0, CHUNKS_PER_TILE, chunk_body, 0)
    plsc.subcore_barrier()
    for q in range(ACC_PER_TILE // DUMP):
        r0 = s * ACC_PER_TILE + q * DUMP
        pltpu.sync_copy(acc_s.at[pl.ds(r0, DUMP), :], stage_v)
        pltpu.sync_copy(stage_v, part_hbm.at[c, pl.ds(r0, DUMP), :])


def _tc_dinv(degp):
    def body(p_ref, o_ref):
        deg = p_ref[0] + p_ref[1] + 1.0
        o_ref[...] = lax.rsqrt(deg)

    return pl.pallas_call(
        body, out_shape=jax.ShapeDtypeStruct((NPAD // C, C), jnp.float32)
    )(degp)


def _tc_mm(x, W, dinv_col):
    def body(x_ref, w_ref, d_ref, o_ref):
        h = jnp.dot(x_ref[...], w_ref[...], preferred_element_type=jnp.float32)
        o_ref[...] = h * d_ref[...]

    return pl.pallas_call(
        body, out_shape=jax.ShapeDtypeStruct((N, C), jnp.float32)
    )(x, W, dinv_col)


def _tc_epi(part, hs, dinv_col, bias, gamma, beta, mean, var):
    def body(p_ref, h_ref, d_ref, b_ref, g_ref, be_ref, m_ref, v_ref, o_ref):
        acc = p_ref[0] + p_ref[1] + h_ref[...] * d_ref[...]
        xx = acc + b_ref[...] - m_ref[...]
        inv = lax.rsqrt(v_ref[...] + 1e-5)
        o_ref[...] = jnp.maximum(xx * inv * g_ref[...] + be_ref[...], 0.0)

    return pl.pallas_call(
        body, out_shape=jax.ShapeDtypeStruct((N, C), jnp.float32)
    )(part, hs, dinv_col, bias, gamma, beta, mean, var)


def kernel(x, edge_index, edge_weight, W, bias, gamma, beta, running_mean, running_var):
    row = edge_index[0].astype(jnp.int32)
    col = edge_index[1].astype(jnp.int32)
    w1 = edge_weight
    degp = _sc_deg(col, w1)
    dinv2d = _tc_dinv(degp.reshape(NC, NPAD // C, C))
    dinv_flat = dinv2d.reshape(NPAD)
    dinv_col = dinv_flat[:N].reshape(N, 1)
    hs = _tc_mm(x, W, dinv_col)
    part = _sc_edge(hs, row, col, w1, dinv_flat)
    return _tc_epi(
        part[:, :N, :], hs, dinv_col,
        bias.reshape(1, C), gamma.reshape(1, C), beta.reshape(1, C),
        running_mean.reshape(1, C), running_var.reshape(1, C),
    )


# speedup vs baseline: 31.3990x; 2.6153x over previous
import functools

import jax
import jax.numpy as jnp
from jax import lax
from jax.experimental import pallas as pl
from jax.experimental.pallas import tpu as pltpu
from jax.experimental.pallas import tpu_sc as plsc

N = 10000
E = 320000
C = 128

NC = 2
NS = 16
NW = NC * NS
KE = 80
EPW = E // NW
CHUNKS = EPW // KE
NBUF = 3
NPAD = 10240
DEG_PER_TILE = NPAD // NS
ACC_PER_TILE = NPAD // NS
DRAIN = 16

_mesh = plsc.VectorSubcoreMesh(core_axis_name="c", subcore_axis_name="s")
_sc_params = pltpu.CompilerParams(needs_layout_passes=False,
                                  use_tc_tiling_on_sc=False)


@functools.partial(
    pl.kernel,
    out_type=jax.ShapeDtypeStruct((NC * NPAD,), jnp.float32),
    mesh=_mesh,
    compiler_params=_sc_params,
    scratch_types=[
        pltpu.VMEM_SHARED((NPAD,), jnp.float32),
        pltpu.VMEM((CHUNKS, KE), jnp.int32),
        pltpu.VMEM((CHUNKS, KE), jnp.float32),
        pltpu.VMEM((DEG_PER_TILE,), jnp.float32),
        pltpu.SemaphoreType.DMA,
    ],
)
def _sc_deg(col_hbm, w_hbm, degp_hbm, deg_s, colv, wv, zv, sem):
    c = lax.axis_index("c")
    s = lax.axis_index("s")
    wid = s * NC + c
    z16 = jnp.zeros((16,), jnp.float32)
    for i in range(DEG_PER_TILE // 16):
        zv[pl.ds(i * 16, 16)] = z16
    pltpu.sync_copy(zv, deg_s.at[pl.ds(s * DEG_PER_TILE, DEG_PER_TILE)])
    pltpu.sync_copy(col_hbm.at[wid], colv)
    pltpu.sync_copy(w_hbm.at[wid], wv)
    plsc.subcore_barrier()

    def drain_one():
        pltpu.make_async_copy(wv.at[0], deg_s.at[colv.at[0]], sem).wait()

    def chunk_body(j, _):
        pltpu.async_copy(wv.at[j], deg_s.at[colv.at[j]], sem, add=True)

        @pl.when(j >= DRAIN)
        def _():
            drain_one()

        return 0

    lax.fori_loop(0, CHUNKS, chunk_body, 0)

    def tail_body(j, _):
        drain_one()
        return 0

    lax.fori_loop(0, DRAIN, tail_body, 0)
    plsc.subcore_barrier()
    pltpu.sync_copy(deg_s.at[pl.ds(s * DEG_PER_TILE, DEG_PER_TILE)], zv)
    pltpu.sync_copy(zv, degp_hbm.at[pl.ds(c * NPAD + s * DEG_PER_TILE, DEG_PER_TILE)])


GROUP = 25
NGROUP = CHUNKS // GROUP


@functools.partial(
    pl.kernel,
    out_type=jax.ShapeDtypeStruct((NC, NPAD, C), jnp.float32),
    mesh=_mesh,
    compiler_params=_sc_params,
    scratch_types=[
        pltpu.VMEM_SHARED((NPAD, C), jnp.float32),
        pltpu.VMEM((NPAD,), jnp.float32),
        pltpu.VMEM((GROUP, KE), jnp.int32),
        pltpu.VMEM((GROUP, KE), jnp.int32),
        pltpu.VMEM((GROUP, KE), jnp.float32),
        [pltpu.VMEM((KE, C), jnp.float32)] * NBUF,
        [pltpu.SemaphoreType.DMA] * NBUF,
        [pltpu.SemaphoreType.DMA] * NBUF,
    ],
)
def _sc_edge(hs_hbm, row_hbm, col_hbm, w_hbm, dinv_hbm, part_hbm,
             acc_s, dinv_v, rowg, colg, wg, rows, gsem, ssem):
    c = lax.axis_index("c")
    s = lax.axis_index("s")
    wid = s * NC + c
    z16 = jnp.zeros((16,), jnp.float32)

    def zrow(i, _):
        for k8 in range(C // 16):
            rows[0][i, pl.ds(k8 * 16, 16)] = z16
        return 0

    lax.fori_loop(0, KE, zrow, 0)
    for q in range(ACC_PER_TILE // KE):
        pltpu.sync_copy(rows[0], acc_s.at[pl.ds(s * ACC_PER_TILE + q * KE, KE), :])
    pltpu.sync_copy(dinv_hbm, dinv_v)
    plsc.subcore_barrier()

    def gather(j, b):
        pltpu.async_copy(hs_hbm.at[rowg.at[j]], rows[b], gsem[b])

    def gwait(b):
        pltpu.make_async_copy(hs_hbm.at[rowg.at[0]], rows[b], gsem[b]).wait()

    def scatter(j, b):
        pltpu.async_copy(rows[b], acc_s.at[colg.at[j]], ssem[b], add=True)

    def swait(b):
        pltpu.make_async_copy(rows[b], acc_s.at[colg.at[0]], ssem[b]).wait()

    def scale(j, b):
        def body(e, _):
            sc = plsc.load_gather(wg, [jnp.full((16,), j, jnp.int32),
                                       jnp.full((16,), e, jnp.int32)])
            for k8 in range(C // 16):
                v = rows[b][e, pl.ds(k8 * 16, 16)]
                rows[b][e, pl.ds(k8 * 16, 16)] = v * sc
            return 0

        lax.fori_loop(0, KE, body, 0)

    def group_body(g, _):
        g0 = g * GROUP
        pltpu.sync_copy(row_hbm.at[wid, pl.ds(g0, GROUP), :], rowg)
        pltpu.sync_copy(col_hbm.at[wid, pl.ds(g0, GROUP), :], colg)
        pltpu.sync_copy(w_hbm.at[wid, pl.ds(g0, GROUP), :], wg)
        for b in range(NBUF):
            gather(b, b)

        def norm_body(j, _):
            for t in range(KE // 16):
                c16 = colg[j, pl.ds(t * 16, 16)]
                w16 = wg[j, pl.ds(t * 16, 16)]
                wg[j, pl.ds(t * 16, 16)] = plsc.load_gather(dinv_v, [c16]) * w16
            return 0

        lax.fori_loop(0, GROUP, norm_body, 0)

        for k in range(GROUP):
            b = k % NBUF
            gwait(b)
            nk = k + 1
            if NBUF <= nk < GROUP:
                swait(nk % NBUF)
                gather(nk, nk % NBUF)
            scale(k, b)
            scatter(k, b)
        for b in range(NBUF):
            swait(b)
        return 0

    lax.fori_loop(0, NGROUP, group_body, 0)
    plsc.subcore_barrier()
    for q in range(ACC_PER_TILE // KE):
        r0 = s * ACC_PER_TILE + q * KE
        pltpu.sync_copy(acc_s.at[pl.ds(r0, KE), :], rows[0])
        pltpu.sync_copy(rows[0], part_hbm.at[c, pl.ds(r0, KE), :])


def _tc_dinv(degp):
    def body(p_ref, o_ref):
        deg = p_ref[0] + p_ref[1] + 1.0
        o_ref[...] = lax.rsqrt(deg)

    return pl.pallas_call(
        body, out_shape=jax.ShapeDtypeStruct((NPAD // C, C), jnp.float32)
    )(degp)


def _tc_mm(x, W, dinv_col):
    def body(x_ref, w_ref, d_ref, o_ref):
        h = jnp.dot(x_ref[...], w_ref[...], preferred_element_type=jnp.float32)
        o_ref[...] = h * d_ref[...]

    return pl.pallas_call(
        body, out_shape=jax.ShapeDtypeStruct((N, C), jnp.float32)
    )(x, W, dinv_col)


def _tc_epi(part, hs, dinv_col, bias, gamma, beta, mean, var):
    def body(p_ref, h_ref, d_ref, b_ref, g_ref, be_ref, m_ref, v_ref, o_ref):
        acc = p_ref[0] + p_ref[1] + h_ref[...] * d_ref[...]
        xx = acc + b_ref[...] - m_ref[...]
        inv = lax.rsqrt(v_ref[...] + 1e-5)
        o_ref[...] = jnp.maximum(xx * inv * g_ref[...] + be_ref[...], 0.0)

    return pl.pallas_call(
        body, out_shape=jax.ShapeDtypeStruct((N, C), jnp.float32)
    )(part, hs, dinv_col, bias, gamma, beta, mean, var)


def kernel(x, edge_index, edge_weight, W, bias, gamma, beta, running_mean, running_var):
    row = edge_index[0].astype(jnp.int32).reshape(NW, CHUNKS, KE)
    col = edge_index[1].astype(jnp.int32).reshape(NW, CHUNKS, KE)
    w3 = edge_weight.reshape(NW, CHUNKS, KE)
    degp = _sc_deg(col, w3)
    dinv2d = _tc_dinv(degp.reshape(NC, NPAD // C, C))
    dinv_flat = dinv2d.reshape(NPAD)
    dinv_col = dinv_flat[:N].reshape(N, 1)
    hs = _tc_mm(x, W, dinv_col)
    part = _sc_edge(hs, row, col, w3, dinv_flat)
    return _tc_epi(
        part[:, :N, :], hs, dinv_col,
        bias.reshape(1, C), gamma.reshape(1, C), beta.reshape(1, C),
        running_mean.reshape(1, C), running_var.reshape(1, C),
    )


# speedup vs baseline: 32.0796x; 1.0217x over previous
import functools

import jax
import jax.numpy as jnp
from jax import lax
from jax.experimental import pallas as pl
from jax.experimental.pallas import tpu as pltpu
from jax.experimental.pallas import tpu_sc as plsc

N = 10000
E = 320000
C = 128

NC = 2
NS = 16
NW = NC * NS
KE = 80
EPW = E // NW
CHUNKS = EPW // KE
NBUF = 3
NPAD = 10240
DEG_PER_TILE = NPAD // NS
ACC_PER_TILE = NPAD // NS
DRAIN = 16
DCHUNKS = E // NS // KE
GROUP = 25
NGROUP = CHUNKS // GROUP

_mesh = plsc.VectorSubcoreMesh(core_axis_name="c", subcore_axis_name="s")
_sc_params = pltpu.CompilerParams(needs_layout_passes=False,
                                  use_tc_tiling_on_sc=False)


def _rsqrt16(v):
    i = plsc.bitcast(v, jnp.int32)
    i = jnp.int32(0x5F3759DF) - lax.shift_right_arithmetic(i, jnp.int32(1))
    y = plsc.bitcast(i, jnp.float32)
    half = v * 0.5
    for _ in range(4):
        y = y * (1.5 - half * y * y)
    return y


@functools.partial(
    pl.kernel,
    out_type=jax.ShapeDtypeStruct((NPAD,), jnp.float32),
    mesh=_mesh,
    compiler_params=_sc_params,
    scratch_types=[
        pltpu.VMEM_SHARED((NPAD,), jnp.float32),
        pltpu.VMEM((DCHUNKS, KE), jnp.int32),
        pltpu.VMEM((DCHUNKS, KE), jnp.float32),
        pltpu.VMEM((DEG_PER_TILE,), jnp.float32),
        pltpu.SemaphoreType.DMA,
    ],
)
def _sc_degdinv(col_hbm, w_hbm, dinv_hbm, deg_s, colv, wv, zv, sem):
    c = lax.axis_index("c")
    s = lax.axis_index("s")
    z16 = jnp.zeros((16,), jnp.float32)
    for i in range(DEG_PER_TILE // 16):
        zv[pl.ds(i * 16, 16)] = z16
    pltpu.sync_copy(zv, deg_s.at[pl.ds(s * DEG_PER_TILE, DEG_PER_TILE)])
    pltpu.sync_copy(col_hbm.at[s], colv)
    pltpu.sync_copy(w_hbm.at[s], wv)
    plsc.subcore_barrier()

    def drain_one():
        pltpu.make_async_copy(wv.at[0], deg_s.at[colv.at[0]], sem).wait()

    def chunk_body(j, _):
        pltpu.async_copy(wv.at[j], deg_s.at[colv.at[j]], sem, add=True)

        @pl.when(j >= DRAIN)
        def _():
            drain_one()

        return 0

    lax.fori_loop(0, DCHUNKS, chunk_body, 0)

    def tail_body(j, _):
        drain_one()
        return 0

    lax.fori_loop(0, DRAIN, tail_body, 0)
    plsc.subcore_barrier()
    pltpu.sync_copy(deg_s.at[pl.ds(s * DEG_PER_TILE, DEG_PER_TILE)], zv)
    for i in range(DEG_PER_TILE // 16):
        zv[pl.ds(i * 16, 16)] = _rsqrt16(zv[pl.ds(i * 16, 16)] + 1.0)

    @pl.when(c == 0)
    def _():
        pltpu.sync_copy(zv, dinv_hbm.at[pl.ds(s * DEG_PER_TILE, DEG_PER_TILE)])


@functools.partial(
    pl.kernel,
    out_type=jax.ShapeDtypeStruct((NC, NPAD, C), jnp.float32),
    mesh=_mesh,
    compiler_params=_sc_params,
    scratch_types=[
        pltpu.VMEM_SHARED((NPAD, C), jnp.float32),
        pltpu.VMEM((NPAD,), jnp.float32),
        pltpu.VMEM((GROUP, KE), jnp.int32),
        pltpu.VMEM((GROUP, KE), jnp.int32),
        pltpu.VMEM((GROUP, KE), jnp.float32),
        [pltpu.VMEM((KE, C), jnp.float32)] * NBUF,
        [pltpu.SemaphoreType.DMA] * NBUF,
        [pltpu.SemaphoreType.DMA] * NBUF,
    ],
)
def _sc_edge(h_hbm, row_hbm, col_hbm, w_hbm, dinv_hbm, part_hbm,
             acc_s, dinv_v, rowg, colg, wg, rows, gsem, ssem):
    c = lax.axis_index("c")
    s = lax.axis_index("s")
    wid = s * NC + c
    z16 = jnp.zeros((16,), jnp.float32)

    def zrow(i, _):
        for k8 in range(C // 16):
            rows[0][i, pl.ds(k8 * 16, 16)] = z16
        return 0

    lax.fori_loop(0, KE, zrow, 0)
    for q in range(ACC_PER_TILE // KE):
        pltpu.sync_copy(rows[0], acc_s.at[pl.ds(s * ACC_PER_TILE + q * KE, KE), :])
    pltpu.sync_copy(dinv_hbm, dinv_v)
    plsc.subcore_barrier()

    def gather(j, b):
        pltpu.async_copy(h_hbm.at[rowg.at[j]], rows[b], gsem[b])

    def gwait(b):
        pltpu.make_async_copy(h_hbm.at[rowg.at[0]], rows[b], gsem[b]).wait()

    def scatter(j, b):
        pltpu.async_copy(rows[b], acc_s.at[colg.at[j]], ssem[b], add=True)

    def swait(b):
        pltpu.make_async_copy(rows[b], acc_s.at[colg.at[0]], ssem[b]).wait()

    def scale1(j, e, b):
        sc = plsc.load_gather(wg, [jnp.full((16,), j, jnp.int32),
                                   jnp.full((16,), e, jnp.int32)])
        for k8 in range(C // 16):
            v = rows[b][e, pl.ds(k8 * 16, 16)]
            rows[b][e, pl.ds(k8 * 16, 16)] = v * sc

    def scale(j, b):
        def body(e2, _):
            e = e2 * 2
            scale1(j, e, b)
            scale1(j, e + 1, b)
            return 0

        lax.fori_loop(0, KE // 2, body, 0)

    def group_body(g, _):
        g0 = g * GROUP
        pltpu.sync_copy(row_hbm.at[wid, pl.ds(g0, GROUP), :], rowg)
        pltpu.sync_copy(col_hbm.at[wid, pl.ds(g0, GROUP), :], colg)
        pltpu.sync_copy(w_hbm.at[wid, pl.ds(g0, GROUP), :], wg)
        for b in range(NBUF):
            gather(b, b)

        def norm_body(j, _):
            for t in range(KE // 16):
                r16 = rowg[j, pl.ds(t * 16, 16)]
                c16 = colg[j, pl.ds(t * 16, 16)]
                w16 = wg[j, pl.ds(t * 16, 16)]
                nv = plsc.load_gather(dinv_v, [r16]) * w16
                wg[j, pl.ds(t * 16, 16)] = nv * plsc.load_gather(dinv_v, [c16])
            return 0

        lax.fori_loop(0, GROUP, norm_body, 0)

        for k in range(GROUP):
            b = k % NBUF
            gwait(b)
            nk = k + 1
            if NBUF <= nk < GROUP:
                swait(nk % NBUF)
                gather(nk, nk % NBUF)
            scale(k, b)
            scatter(k, b)
        for b in range(NBUF):
            swait(b)
        return 0

    lax.fori_loop(0, NGROUP, group_body, 0)
    plsc.subcore_barrier()
    for q in range(ACC_PER_TILE // KE):
        r0 = s * ACC_PER_TILE + q * KE
        pltpu.sync_copy(acc_s.at[pl.ds(r0, KE), :], rows[0])
        pltpu.sync_copy(rows[0], part_hbm.at[c, pl.ds(r0, KE), :])


def _tc_mm(x, W):
    def body(x_ref, w_ref, o_ref):
        o_ref[...] = jnp.dot(x_ref[...], w_ref[...],
                             preferred_element_type=jnp.float32)

    return pl.pallas_call(
        body, out_shape=jax.ShapeDtypeStruct((N, C), jnp.float32)
    )(x, W)


def _tc_epi(part, h, dinv_col, bias, gamma, beta, mean, var):
    def body(p_ref, h_ref, d_ref, b_ref, g_ref, be_ref, m_ref, v_ref, o_ref):
        d = d_ref[...]
        acc = p_ref[0] + p_ref[1] + h_ref[...] * (d * d)
        xx = acc + b_ref[...] - m_ref[...]
        inv = lax.rsqrt(v_ref[...] + 1e-5)
        o_ref[...] = jnp.maximum(xx * inv * g_ref[...] + be_ref[...], 0.0)

    return pl.pallas_call(
        body, out_shape=jax.ShapeDtypeStruct((N, C), jnp.float32)
    )(part, h, dinv_col, bias, gamma, beta, mean, var)


def kernel(x, edge_index, edge_weight, W, bias, gamma, beta, running_mean, running_var):
    row = edge_index[0].astype(jnp.int32).reshape(NW, CHUNKS, KE)
    col = edge_index[1].astype(jnp.int32).reshape(NW, CHUNKS, KE)
    w3 = edge_weight.reshape(NW, CHUNKS, KE)
    col_d = edge_index[1].astype(jnp.int32).reshape(NS, DCHUNKS, KE)
    w_d = edge_weight.reshape(NS, DCHUNKS, KE)
    dinv_flat = _sc_degdinv(col_d, w_d)
    dinv_col = dinv_flat[:N].reshape(N, 1)
    h = _tc_mm(x, W)
    part = _sc_edge(h, row, col, w3, dinv_flat)
    return _tc_epi(
        part[:, :N, :], h, dinv_col,
        bias.reshape(1, C), gamma.reshape(1, C), beta.reshape(1, C),
        running_mean.reshape(1, C), running_var.reshape(1, C),
    )


# speedup vs baseline: 33.8563x; 1.0554x over previous
import functools

import jax
import jax.numpy as jnp
from jax import lax
from jax.experimental import pallas as pl
from jax.experimental.pallas import tpu as pltpu
from jax.experimental.pallas import tpu_sc as plsc

N = 10000
E = 320000
C = 128

NC = 2
NS = 16
NW = NC * NS
KE = 80
EPW = E // NW
CHUNKS = EPW // KE
NBUF = 3
NPAD = 10240
DEG_PER_TILE = NPAD // NS
ACC_PER_TILE = NPAD // NS
DRAIN = 16
DCHUNKS = E // NS // KE
GROUP = 25
NGROUP = CHUNKS // GROUP

_mesh = plsc.VectorSubcoreMesh(core_axis_name="c", subcore_axis_name="s")
_sc_params = pltpu.CompilerParams(needs_layout_passes=False,
                                  use_tc_tiling_on_sc=False)


def _rsqrt16(v):
    i = plsc.bitcast(v, jnp.int32)
    i = jnp.int32(0x5F3759DF) - lax.shift_right_arithmetic(i, jnp.int32(1))
    y = plsc.bitcast(i, jnp.float32)
    half = v * 0.5
    for _ in range(4):
        y = y * (1.5 - half * y * y)
    return y


@functools.partial(
    pl.kernel,
    out_type=jax.ShapeDtypeStruct((NPAD,), jnp.float32),
    mesh=_mesh,
    compiler_params=_sc_params,
    scratch_types=[
        pltpu.VMEM_SHARED((NPAD,), jnp.float32),
        pltpu.VMEM((DCHUNKS, KE), jnp.int32),
        pltpu.VMEM((DCHUNKS, KE), jnp.float32),
        pltpu.VMEM((DEG_PER_TILE,), jnp.float32),
        pltpu.SemaphoreType.DMA,
    ],
)
def _sc_degdinv(col_hbm, w_hbm, dinv_hbm, deg_s, colv, wv, zv, sem):
    c = lax.axis_index("c")
    s = lax.axis_index("s")
    z16 = jnp.zeros((16,), jnp.float32)
    for i in range(DEG_PER_TILE // 16):
        zv[pl.ds(i * 16, 16)] = z16
    pltpu.sync_copy(zv, deg_s.at[pl.ds(s * DEG_PER_TILE, DEG_PER_TILE)])
    pltpu.sync_copy(col_hbm.at[s], colv)
    pltpu.sync_copy(w_hbm.at[s], wv)
    plsc.subcore_barrier()

    def drain_one():
        pltpu.make_async_copy(wv.at[0], deg_s.at[colv.at[0]], sem).wait()

    def chunk_body(j, _):
        pltpu.async_copy(wv.at[j], deg_s.at[colv.at[j]], sem, add=True)

        @pl.when(j >= DRAIN)
        def _():
            drain_one()

        return 0

    lax.fori_loop(0, DCHUNKS, chunk_body, 0)

    def tail_body(j, _):
        drain_one()
        return 0

    lax.fori_loop(0, DRAIN, tail_body, 0)
    plsc.subcore_barrier()
    pltpu.sync_copy(deg_s.at[pl.ds(s * DEG_PER_TILE, DEG_PER_TILE)], zv)
    for i in range(DEG_PER_TILE // 16):
        zv[pl.ds(i * 16, 16)] = _rsqrt16(zv[pl.ds(i * 16, 16)] + 1.0)

    @pl.when(c == 0)
    def _():
        pltpu.sync_copy(zv, dinv_hbm.at[pl.ds(s * DEG_PER_TILE, DEG_PER_TILE)])


@functools.partial(
    pl.kernel,
    out_type=jax.ShapeDtypeStruct((NC, NPAD, C), jnp.float32),
    mesh=_mesh,
    compiler_params=_sc_params,
    scratch_types=[
        pltpu.VMEM_SHARED((NPAD, C), jnp.float32),
        pltpu.VMEM((NPAD,), jnp.float32),
        pltpu.VMEM((GROUP, 2, KE // 2), jnp.int32),
        pltpu.VMEM((GROUP, KE), jnp.int32),
        pltpu.VMEM((GROUP, KE), jnp.float32),
        [pltpu.VMEM((KE, C), jnp.float32)] * NBUF,
        [pltpu.SemaphoreType.DMA] * NBUF,
        [pltpu.SemaphoreType.DMA] * NBUF,
        [pltpu.SemaphoreType.DMA] * NBUF,
    ],
)
def _sc_edge(h_hbm, row_hbm, col_hbm, w_hbm, dinv_hbm, part_hbm,
             acc_s, dinv_v, rowg, colg, wg, rows, gsemA, gsemB, ssem):
    c = lax.axis_index("c")
    s = lax.axis_index("s")
    wid = s * NC + c
    z16 = jnp.zeros((16,), jnp.float32)
    KH = KE // 2

    def zrow(i, _):
        for k8 in range(C // 16):
            rows[0][i, pl.ds(k8 * 16, 16)] = z16
        return 0

    lax.fori_loop(0, KE, zrow, 0)
    for q in range(ACC_PER_TILE // KE):
        pltpu.sync_copy(rows[0], acc_s.at[pl.ds(s * ACC_PER_TILE + q * KE, KE), :])
    pltpu.sync_copy(dinv_hbm, dinv_v)
    plsc.subcore_barrier()

    def gather(j, b):
        pltpu.async_copy(h_hbm.at[rowg.at[j, 0]],
                         rows[b].at[pl.ds(0, KH), :], gsemA[b])
        pltpu.async_copy(h_hbm.at[rowg.at[j, 1]],
                         rows[b].at[pl.ds(KH, KH), :], gsemB[b])

    def gwait(b):
        pltpu.make_async_copy(h_hbm.at[rowg.at[0, 0]],
                              rows[b].at[pl.ds(0, KH), :], gsemA[b]).wait()
        pltpu.make_async_copy(h_hbm.at[rowg.at[0, 0]],
                              rows[b].at[pl.ds(KH, KH), :], gsemB[b]).wait()

    def scatter(j, b):
        pltpu.async_copy(rows[b], acc_s.at[colg.at[j]], ssem[b], add=True)

    def swait(b):
        pltpu.make_async_copy(rows[b], acc_s.at[colg.at[0]], ssem[b]).wait()

    def scale1(j, e, b):
        sc = plsc.load_gather(wg, [jnp.full((16,), j, jnp.int32),
                                   jnp.full((16,), e, jnp.int32)])
        for k8 in range(C // 16):
            v = rows[b][e, pl.ds(k8 * 16, 16)]
            rows[b][e, pl.ds(k8 * 16, 16)] = v * sc

    def scale(j, b):
        def body(e2, _):
            e = e2 * 2
            scale1(j, e, b)
            scale1(j, e + 1, b)
            return 0

        lax.fori_loop(0, KE // 2, body, 0)

    iota16 = lax.iota(jnp.int32, 16)

    def group_body(g, _):
        g0 = g * GROUP
        pltpu.sync_copy(row_hbm.at[wid, pl.ds(g0, GROUP), :, :], rowg)
        pltpu.sync_copy(col_hbm.at[wid, pl.ds(g0, GROUP), :], colg)
        pltpu.sync_copy(w_hbm.at[wid, pl.ds(g0, GROUP), :], wg)
        for b in range(NBUF):
            gather(b, b)

        def norm_body(j, _):
            j16 = jnp.full((16,), j, jnp.int32)
            for t in range(KE // 16):
                e16 = iota16 + (t * 16)
                h16 = e16 // KH
                r16 = plsc.load_gather(rowg, [j16, h16, e16 - h16 * KH])
                c16 = colg[j, pl.ds(t * 16, 16)]
                w16 = wg[j, pl.ds(t * 16, 16)]
                nv = plsc.load_gather(dinv_v, [r16]) * w16
                wg[j, pl.ds(t * 16, 16)] = nv * plsc.load_gather(dinv_v, [c16])
            return 0

        lax.fori_loop(0, GROUP, norm_body, 0)

        for k in range(GROUP):
            b = k % NBUF
            gwait(b)
            nk = k + 1
            if NBUF <= nk < GROUP:
                swait(nk % NBUF)
                gather(nk, nk % NBUF)
            scale(k, b)
            scatter(k, b)
        for b in range(NBUF):
            swait(b)
        return 0

    lax.fori_loop(0, NGROUP, group_body, 0)
    plsc.subcore_barrier()
    for q in range(ACC_PER_TILE // KE):
        r0 = s * ACC_PER_TILE + q * KE
        pltpu.sync_copy(acc_s.at[pl.ds(r0, KE), :], rows[0])
        pltpu.sync_copy(rows[0], part_hbm.at[c, pl.ds(r0, KE), :])


_RB = 2000


def _tc_mm(x, W):
    def body(x_ref, w_ref, o_ref):
        o_ref[...] = jnp.dot(x_ref[...], w_ref[...],
                             preferred_element_type=jnp.float32)

    return pl.pallas_call(
        body,
        grid=(N // _RB,),
        in_specs=[pl.BlockSpec((_RB, C), lambda i: (i, 0)),
                  pl.BlockSpec((C, C), lambda i: (0, 0))],
        out_specs=pl.BlockSpec((_RB, C), lambda i: (i, 0)),
        out_shape=jax.ShapeDtypeStruct((N, C), jnp.float32),
    )(x, W)


def _tc_epi(part, h, dinv_col, bias, gamma, beta, mean, var):
    def body(p_ref, h_ref, d_ref, b_ref, g_ref, be_ref, m_ref, v_ref, o_ref):
        d = d_ref[...]
        acc = p_ref[0] + p_ref[1] + h_ref[...] * (d * d)
        xx = acc + b_ref[...] - m_ref[...]
        inv = lax.rsqrt(v_ref[...] + 1e-5)
        o_ref[...] = jnp.maximum(xx * inv * g_ref[...] + be_ref[...], 0.0)

    vec = pl.BlockSpec((1, C), lambda i: (0, 0))
    return pl.pallas_call(
        body,
        grid=(N // _RB,),
        in_specs=[pl.BlockSpec((NC, _RB, C), lambda i: (0, i, 0)),
                  pl.BlockSpec((_RB, C), lambda i: (i, 0)),
                  pl.BlockSpec((_RB, 1), lambda i: (i, 0)),
                  vec, vec, vec, vec, vec],
        out_specs=pl.BlockSpec((_RB, C), lambda i: (i, 0)),
        out_shape=jax.ShapeDtypeStruct((N, C), jnp.float32),
    )(part, h, dinv_col, bias, gamma, beta, mean, var)


def kernel(x, edge_index, edge_weight, W, bias, gamma, beta, running_mean, running_var):
    row = edge_index[0].astype(jnp.int32).reshape(NW, CHUNKS, KE)
    col = edge_index[1].astype(jnp.int32).reshape(NW, CHUNKS, KE)
    w3 = edge_weight.reshape(NW, CHUNKS, KE)
    col_d = edge_index[1].astype(jnp.int32).reshape(NS, DCHUNKS, KE)
    w_d = edge_weight.reshape(NS, DCHUNKS, KE)
    dinv_flat = _sc_degdinv(col_d, w_d)
    dinv_col = dinv_flat[:N].reshape(N, 1)
    h = _tc_mm(x, W)
    part = _sc_edge(h, row.reshape(NW, CHUNKS, 2, KE // 2), col, w3, dinv_flat)
    return _tc_epi(
        part, h, dinv_col,
        bias.reshape(1, C), gamma.reshape(1, C), beta.reshape(1, C),
        running_mean.reshape(1, C), running_var.reshape(1, C),
    )
